# baseline (speedup 1.0000x reference)
import functools

import jax
import jax.numpy as jnp
from jax import lax
from jax.experimental import pallas as pl
from jax.experimental.pallas import tpu as pltpu
from jax.experimental.pallas import tpu_sc as plsc

N = 10000
D = 256
DH = 128
E = 160000
N_PAD = 10240
NH = 5120
NA = 5248
RPT_A = 2 * NA // 16
E_PAD = 163840
EPT = E_PAD // 16
GCH = 64
EPW = E_PAD // 32
NCW = EPW // 128
RPT_N = N_PAD // 16
STAGE = 40
NSTG = EPT // (GCH * STAGE)

_MESH = plsc.VectorSubcoreMesh(core_axis_name="c", subcore_axis_name="s")


def _agg_body(h, src2i, dst2i, zeros, agg,
              acc, src_v, dst_v, rows0, rows1, g0, g1):
  cid = lax.axis_index("c")
  sid = lax.axis_index("s")
  rows = pl.ds(sid * RPT_A, RPT_A)
  pltpu.sync_copy(zeros, acc.at[rows])
  plsc.subcore_barrier()

  def do_stage(st):
    pltpu.sync_copy(
        src2i.at[pl.ds(sid * (NSTG * STAGE) + st * STAGE, STAGE)], src_v)
    pltpu.sync_copy(
        dst2i.at[pl.ds(cid * (E_PAD // GCH) + sid * (NSTG * STAGE)
                       + st * STAGE, STAGE)], dst_v)
    pltpu.async_copy(h.at[src_v.at[0]], rows0, g0)

    def step(k, carry):
      i = 2 * k
      pltpu.async_copy(h.at[src_v.at[i + 1]], rows1, g1)
      pltpu.make_async_copy(h.at[src_v.at[0]], rows0, g0).wait()
      pltpu.sync_copy(rows0, acc.at[dst_v.at[i]], add=True)

      @pl.when(k < STAGE // 2 - 1)
      def _():
        pltpu.async_copy(h.at[src_v.at[i + 2]], rows0, g0)

      pltpu.make_async_copy(h.at[src_v.at[0]], rows1, g1).wait()
      pltpu.sync_copy(rows1, acc.at[dst_v.at[i + 1]], add=True)
      return carry

    lax.fori_loop(0, STAGE // 2, step, 0)

  for st in range(NSTG):
    do_stage(st)
  plsc.subcore_barrier()
  pltpu.sync_copy(acc.at[rows], agg.at[cid].at[rows])


_agg_call = pl.kernel(
    _agg_body,
    out_type=jax.ShapeDtypeStruct((2, 2 * NA, DH), jnp.float32),
    mesh=_MESH,
    scratch_types=[
        pltpu.VMEM_SHARED((2 * NA, DH), jnp.float32),
        pltpu.VMEM((STAGE, 128), jnp.int32),
        pltpu.VMEM((STAGE, 128), jnp.int32),
        pltpu.VMEM((128, DH), jnp.float32),
        pltpu.VMEM((128, DH), jnp.float32),
        pltpu.SemaphoreType.DMA,
        pltpu.SemaphoreType.DMA,
    ],
)


def _cnt_body(dst, zeros, ones, cnt, acc, dst_v, ones_v, sem):
  cid = lax.axis_index("c")
  sid = lax.axis_index("s")
  rows = pl.ds(sid * RPT_N, RPT_N)
  pltpu.sync_copy(zeros, acc.at[rows])
  pltpu.sync_copy(ones, ones_v)
  pltpu.sync_copy(dst.at[pl.ds((cid * 16 + sid) * NCW, NCW)], dst_v)
  plsc.subcore_barrier()

  def step(i, carry):
    pltpu.sync_copy(ones_v, acc.at[dst_v.at[i]], add=True)
    return carry

  lax.fori_loop(0, NCW, step, 0)
  plsc.subcore_barrier()
  pltpu.sync_copy(acc.at[rows], cnt.at[cid].at[rows])


_cnt_call = pl.kernel(
    _cnt_body,
    out_type=jax.ShapeDtypeStruct((2, N_PAD, 128), jnp.float32),
    mesh=_MESH,
    scratch_types=[
        pltpu.VMEM_SHARED((N_PAD, 128), jnp.float32),
        pltpu.VMEM((NCW, 128), jnp.int32),
        pltpu.VMEM((128, 128), jnp.float32),
        pltpu.SemaphoreType.DMA,
    ],
)


BN = 1024


def _dense_body(act, agg, h, cnt, wl, wr, b, out):
  c = cnt[0, :, 0:1] + cnt[1, :, 0:1]
  inv = 1.0 / jnp.clip(c, 1.0, None)
  a = jnp.dot(agg[0, :, 0, :] * inv, wl[0],
              preferred_element_type=jnp.float32)
  a += jnp.dot(agg[0, :, 1, :] * inv, wl[1],
               preferred_element_type=jnp.float32)
  a += jnp.dot(h[:, 0, :], wr[0], preferred_element_type=jnp.float32)
  a += jnp.dot(h[:, 1, :], wr[1], preferred_element_type=jnp.float32)
  a += b[...]
  if act == "relu":
    res = jnp.maximum(a, 0.0)
    out[:, 0, :] = res[:, :DH]
    out[:, 1, :] = res[:, DH:]
  else:
    m = jnp.max(a, axis=-1, keepdims=True)
    z = a - m
    out[...] = z - jnp.log(jnp.sum(jnp.exp(z), axis=-1, keepdims=True))


def _make_dense(act):
  if act == "relu":
    out_spec = pl.BlockSpec((BN, 2, DH), lambda i: (i, 0, 0))
    out_shape = jax.ShapeDtypeStruct((N_PAD, 2, DH), jnp.float32)
  else:
    out_spec = pl.BlockSpec((BN, D), lambda i: (i, 0))
    out_shape = jax.ShapeDtypeStruct((N_PAD, D), jnp.float32)
  return pl.pallas_call(
      functools.partial(_dense_body, act),
      grid=(N_PAD // BN,),
      in_specs=[
          pl.BlockSpec((1, BN, 2, DH), lambda i: (i // 5, i % 5, 0, 0)),
          pl.BlockSpec((BN, 2, DH), lambda i: (i, 0, 0)),
          pl.BlockSpec((2, BN, 128), lambda i: (0, i, 0)),
          pl.BlockSpec((2, DH, D), lambda i: (0, 0, 0)),
          pl.BlockSpec((2, DH, D), lambda i: (0, 0, 0)),
          pl.BlockSpec((1, D), lambda i: (0, 0)),
      ],
      out_specs=out_spec,
      out_shape=out_shape,
  )


_dense_relu = _make_dense("relu")
_dense_lsm = _make_dense("lsm")


def _interleave(idx):
  return jnp.stack([2 * idx, 2 * idx + 1], axis=-1).reshape(-1)


@jax.jit
def kernel(x, edge_index, W_l0, b_l0, W_r0, W_l1, b_l1, W_r1, W_l2, b_l2,
           W_r2):
  xp = jnp.zeros((N_PAD, D), jnp.float32).at[:N].set(x)
  h = xp.reshape(2 * N_PAD, DH)
  src = edge_index[0].astype(jnp.int32)
  dst = edge_index[1].astype(jnp.int32)
  pad = jnp.full((E_PAD - E,), N, jnp.int32)
  src = jnp.concatenate([src, pad])
  dst = jnp.concatenate([dst, pad])
  src2i = _interleave(src).reshape(-1, 128)
  e_idx = jnp.arange(E_PAD, dtype=jnp.int32)
  garb = NH + 1 + (e_idx % 127)
  d0 = jnp.where(dst < NH, dst, garb)
  d1 = jnp.where(dst >= NH, dst - NH, garb)
  dst2i = jnp.concatenate(
      [_interleave(d0), _interleave(d1)]).reshape(-1, 128)
  dstc = dst.reshape(-1, 128)
  zeros_a = jnp.zeros((RPT_A, DH), jnp.float32)
  zeros_n = jnp.zeros((RPT_N, 128), jnp.float32)
  ones = jnp.ones((128, 128), jnp.float32)
  wl = [W.reshape(2, DH, D) for W in (W_l0, W_l1, W_l2)]
  wr = [W.reshape(2, DH, D) for W in (W_r0, W_r1, W_r2)]
  bs = [b.reshape(1, D) for b in (b_l0, b_l1, b_l2)]

  cnt = _cnt_call(dstc, zeros_n, ones)

  def layer(h, w_l, w_r, b, dense):
    agg = _agg_call(h.reshape(2 * N_PAD, DH), src2i, dst2i, zeros_a)
    return dense(agg.reshape(2, NA, 2, DH), h.reshape(N_PAD, 2, DH),
                 cnt, w_l, w_r, b)

  h = layer(h, wl[0], wr[0], bs[0], _dense_relu)
  h = layer(h, wl[1], wr[1], bs[1], _dense_relu)
  out = layer(h, wl[2], wr[2], bs[2], _dense_lsm)

  return out[:N]


# speedup vs baseline: 1.6674x; 1.6674x over previous
import functools

import jax
import jax.numpy as jnp
from jax import lax
from jax.experimental import pallas as pl
from jax.experimental.pallas import tpu as pltpu
from jax.experimental.pallas import tpu_sc as plsc

N = 10000
D = 256
DH = 128
E = 160000
N_PAD = 10240
RPT = N_PAD // 16
CHUNK = 128
E_PAD = 163840
EPT = E_PAD // 16
EPW = E_PAD // 32
NCH = EPT // CHUNK
HALF = NCH // 2
NCW = EPW // CHUNK

_MESH = plsc.VectorSubcoreMesh(core_axis_name="c", subcore_axis_name="s")


def _agg_body(hcat, src2, dst, zeros, agg,
              acc, src_v, dst_v, rows0, rows1, sem0, sem1):
  cid = lax.axis_index("c")
  sid = lax.axis_index("s")
  rows = pl.ds(sid * RPT, RPT)
  pltpu.sync_copy(zeros, acc.at[rows])
  plsc.subcore_barrier()

  def do_half(h):
    pltpu.sync_copy(
        src2.at[pl.ds(cid * (E_PAD // CHUNK) + sid * NCH + h * HALF, HALF)],
        src_v)
    pltpu.sync_copy(dst.at[pl.ds(sid * NCH + h * HALF, HALF)], dst_v)
    pltpu.async_copy(hcat.at[src_v.at[0]], rows0, sem0)

    def step(k, carry):
      i = 2 * k
      pltpu.async_copy(hcat.at[src_v.at[i + 1]], rows1, sem1)
      pltpu.make_async_copy(hcat.at[src_v.at[0]], rows0, sem0).wait()
      pltpu.sync_copy(rows0, acc.at[dst_v.at[i]], add=True)

      @pl.when(k < HALF // 2 - 1)
      def _():
        pltpu.async_copy(hcat.at[src_v.at[i + 2]], rows0, sem0)

      pltpu.make_async_copy(hcat.at[src_v.at[0]], rows1, sem1).wait()
      pltpu.sync_copy(rows1, acc.at[dst_v.at[i + 1]], add=True)
      return carry

    lax.fori_loop(0, HALF // 2, step, 0)

  do_half(0)
  do_half(1)
  plsc.subcore_barrier()
  pltpu.sync_copy(acc.at[rows], agg.at[cid].at[rows])


_agg_call = pl.kernel(
    _agg_body,
    out_type=jax.ShapeDtypeStruct((2, N_PAD, DH), jnp.float32),
    mesh=_MESH,
    scratch_types=[
        pltpu.VMEM_SHARED((N_PAD, DH), jnp.float32),
        pltpu.VMEM((HALF, CHUNK), jnp.int32),
        pltpu.VMEM((HALF, CHUNK), jnp.int32),
        pltpu.VMEM((CHUNK, DH), jnp.float32),
        pltpu.VMEM((CHUNK, DH), jnp.float32),
        pltpu.SemaphoreType.DMA,
        pltpu.SemaphoreType.DMA,
    ],
)


def _cnt_body(dst, zeros, ones, cnt, acc, dst_v, ones_v, sem):
  cid = lax.axis_index("c")
  sid = lax.axis_index("s")
  rows = pl.ds(sid * RPT, RPT)
  pltpu.sync_copy(zeros, acc.at[rows])
  pltpu.sync_copy(ones, ones_v)
  pltpu.sync_copy(dst.at[pl.ds((cid * 16 + sid) * NCW, NCW)], dst_v)
  plsc.subcore_barrier()

  def step(i, carry):
    pltpu.sync_copy(ones_v, acc.at[dst_v.at[i]], add=True)
    return carry

  lax.fori_loop(0, NCW, step, 0)
  plsc.subcore_barrier()
  pltpu.sync_copy(acc.at[rows], cnt.at[cid].at[rows])


_cnt_call = pl.kernel(
    _cnt_body,
    out_type=jax.ShapeDtypeStruct((2, N_PAD, DH), jnp.float32),
    mesh=_MESH,
    scratch_types=[
        pltpu.VMEM_SHARED((N_PAD, DH), jnp.float32),
        pltpu.VMEM((NCW, CHUNK), jnp.int32),
        pltpu.VMEM((CHUNK, DH), jnp.float32),
        pltpu.SemaphoreType.DMA,
    ],
)


BN = 1024


def _dense_body(act, agg, h, cnt, wl, wr, b, out):
  c = cnt[0, :, 0:1] + cnt[1, :, 0:1]
  inv = 1.0 / jnp.clip(c, 1.0, None)
  a = jnp.dot(agg[0] * inv, wl[0], preferred_element_type=jnp.float32)
  a += jnp.dot(agg[1] * inv, wl[1], preferred_element_type=jnp.float32)
  a += jnp.dot(h[0], wr[0], preferred_element_type=jnp.float32)
  a += jnp.dot(h[1], wr[1], preferred_element_type=jnp.float32)
  a += b[...]
  if act == "relu":
    res = jnp.maximum(a, 0.0)
    out[0] = res[:, :DH]
    out[1] = res[:, DH:]
  else:
    m = jnp.max(a, axis=-1, keepdims=True)
    z = a - m
    out[...] = z - jnp.log(jnp.sum(jnp.exp(z), axis=-1, keepdims=True))


def _make_dense(act):
  stack_spec = pl.BlockSpec((2, BN, DH), lambda i: (0, i, 0))
  if act == "relu":
    out_spec = stack_spec
    out_shape = jax.ShapeDtypeStruct((2, N_PAD, DH), jnp.float32)
  else:
    out_spec = pl.BlockSpec((BN, D), lambda i: (i, 0))
    out_shape = jax.ShapeDtypeStruct((N_PAD, D), jnp.float32)
  return pl.pallas_call(
      functools.partial(_dense_body, act),
      grid=(N_PAD // BN,),
      in_specs=[
          stack_spec,
          stack_spec,
          stack_spec,
          pl.BlockSpec((2, DH, D), lambda i: (0, 0, 0)),
          pl.BlockSpec((2, DH, D), lambda i: (0, 0, 0)),
          pl.BlockSpec((1, D), lambda i: (0, 0)),
      ],
      out_specs=out_spec,
      out_shape=out_shape,
  )


_dense_relu = _make_dense("relu")
_dense_lsm = _make_dense("lsm")


@jax.jit
def kernel(x, edge_index, W_l0, b_l0, W_r0, W_l1, b_l1, W_r1, W_l2, b_l2,
           W_r2):
  xp = jnp.zeros((N_PAD, D), jnp.float32).at[:N].set(x)
  h = jnp.stack([xp[:, :DH], xp[:, DH:]])
  src = edge_index[0].astype(jnp.int32)
  dst = edge_index[1].astype(jnp.int32)
  pad = jnp.full((E_PAD - E,), N, jnp.int32)
  src = jnp.concatenate([src, pad])
  dst = jnp.concatenate([dst, pad])
  src2 = jnp.concatenate([src, src + N_PAD]).reshape(-1, CHUNK)
  dst = dst.reshape(-1, CHUNK)
  zeros = jnp.zeros((RPT, DH), jnp.float32)
  ones = jnp.ones((CHUNK, DH), jnp.float32)
  wl = [W.reshape(2, DH, D) for W in (W_l0, W_l1, W_l2)]
  wr = [W.reshape(2, DH, D) for W in (W_r0, W_r1, W_r2)]
  bs = [b.reshape(1, D) for b in (b_l0, b_l1, b_l2)]

  cnt = _cnt_call(dst, zeros, ones)

  agg = _agg_call(h.reshape(2 * N_PAD, DH), src2, dst, zeros)
  h = _dense_relu(agg, h, cnt, wl[0], wr[0], bs[0])
  agg = _agg_call(h.reshape(2 * N_PAD, DH), src2, dst, zeros)
  h = _dense_relu(agg, h, cnt, wl[1], wr[1], bs[1])
  agg = _agg_call(h.reshape(2 * N_PAD, DH), src2, dst, zeros)
  out = _dense_lsm(agg, h, cnt, wl[2], wr[2], bs[2])

  return out[:N]
